# baseline (speedup 1.0000x reference)
import jax
import jax.numpy as jnp
from jax.experimental import pallas as pl
from jax.experimental.pallas import tpu as pltpu

B = 256
MAX_N = 64
NVT = 20
HS = 501
NZ = 56
HP = 512
GP = 3 * HP
NP = 128
ZP = 128
PHASES = 8
PLEN = MAX_N // PHASES


def _body(nt_ref, adjP_ref, wih_ref, whh_ref, bhh_ref, wgm_ref, wgmb_ref,
          w1_ref, b1_ref, w2_ref, b2_ref,
          mu_ref, lv_ref, m_ref, hv_ref):
    m_ref[...] = jnp.zeros_like(m_ref)
    vlane = jax.lax.broadcasted_iota(jnp.int32, (1, MAX_N), 1)

    def make_step(pref):
        def step(v, agg):
            vm1 = jnp.maximum(v - 1, 0)
            c1 = adjP_ref[pl.ds(v, 1), pl.ds(vm1, 1)].reshape(1, B)
            mprev = m_ref[pl.ds(vm1, 1)][0]
            h_in = agg + (mprev * c1).astype(jnp.float32)
            ntv = nt_ref[pl.ds(v, 1)][0]
            onehot = (jax.lax.broadcasted_iota(jnp.int32, (NP, B), 0)
                      == ntv).astype(jnp.bfloat16)
            gi = jnp.dot(wih_ref[...], onehot,
                         preferred_element_type=jnp.float32)
            gh = jnp.dot(whh_ref[...], h_in.astype(jnp.bfloat16),
                         preferred_element_type=jnp.float32) + bhh_ref[...]
            r = jax.nn.sigmoid(gi[0:HP] + gh[0:HP])
            z = jax.nn.sigmoid(gi[HP:2 * HP] + gh[HP:2 * HP])
            n = jnp.tanh(gi[2 * HP:GP] + r * gh[2 * HP:GP])
            hv = (1.0 - z) * n + z * h_in
            hv_ref[...] = hv
            vmask = (vlane == v).astype(jnp.float32)
            gmb = jnp.sum(wgmb_ref[...] * vmask, axis=1, keepdims=True)
            gm = jnp.dot(wgm_ref[...], hv.astype(jnp.bfloat16),
                         preferred_element_type=jnp.float32) + gmb
            mv = jax.nn.sigmoid(gm[0:HP]) * gm[HP:2 * HP]
            vp1 = jnp.minimum(v + 1, MAX_N - 1)
            cn = adjP_ref[pl.ds(vp1, 1)][0, 0:pref]
            prod = (m_ref[0:pref] * cn).reshape(pref // 4, 4, HP, B)
            pairs = ((prod[:, 0] + prod[:, 1]) + (prod[:, 2] + prod[:, 3]))
            agg_next = jnp.sum(pairs, axis=0, dtype=jnp.float32)
            m_ref[pl.ds(v, 1)] = mv.astype(jnp.bfloat16)[None]
            return agg_next
        return step

    agg = jnp.zeros((HP, B), jnp.float32)
    for p in range(PHASES):
        agg = jax.lax.fori_loop(p * PLEN, (p + 1) * PLEN,
                                make_step((p + 1) * PLEN), agg)
    hv = hv_ref[...]
    mu_ref[...] = jnp.dot(w1_ref[...], hv,
                          preferred_element_type=jnp.float32) + b1_ref[...]
    lv_ref[...] = jnp.dot(w2_ref[...], hv,
                          preferred_element_type=jnp.float32) + b2_ref[...]


def _pack3(w, cols, dtype):
    out = jnp.zeros((GP, cols), jnp.float32)
    for k in range(3):
        out = out.at[k * HP:k * HP + HS, :w.shape[1]].set(
            w[k * HS:(k + 1) * HS, :])
    return out.astype(dtype)


def kernel(node_types, adj, gru_Wih, gru_Whh, gru_bih, gru_bhh,
           Wg, bg, Wm, W1, b1, W2, b2):
    f32, bf16 = jnp.float32, jnp.bfloat16
    nt = node_types.astype(jnp.int32).T[:, None, :]
    adjP = jnp.transpose(adj, (2, 1, 0))[:, :, None, :].astype(bf16)

    wih = _pack3(gru_Wih + gru_bih[:, None], NP, bf16)
    whh = _pack3(gru_Whh, HP, bf16)
    bhh = _pack3(gru_bhh[:, None], 1, f32)

    wgm = (jnp.zeros((2 * HP, HP), f32)
           .at[:HS, :HS].set(Wg[:, :HS])
           .at[HP:HP + HS, :HS].set(Wm[:, :HS])).astype(bf16)
    wgmb = (jnp.zeros((2 * HP, MAX_N), f32)
            .at[:HS, :].set(bg[:, None] + Wg[:, HS:])
            .at[HP:HP + HS, :].set(Wm[:, HS:]))

    w1 = jnp.zeros((ZP, HP), f32).at[:NZ, :HS].set(W1)
    b1p = jnp.zeros((ZP, 1), f32).at[:NZ, 0].set(b1)
    w2 = jnp.zeros((ZP, HP), f32).at[:NZ, :HS].set(W2)
    b2p = jnp.zeros((ZP, 1), f32).at[:NZ, 0].set(b2)

    mu, lv = pl.pallas_call(
        _body,
        out_shape=(jax.ShapeDtypeStruct((ZP, B), f32),
                   jax.ShapeDtypeStruct((ZP, B), f32)),
        scratch_shapes=[pltpu.VMEM((MAX_N, HP, B), bf16),
                        pltpu.VMEM((HP, B), f32)],
        compiler_params=pltpu.CompilerParams(
            vmem_limit_bytes=120 * 1024 * 1024),
    )(nt, adjP, wih, whh, bhh, wgm, wgmb, w1, b1p, w2, b2p)
    return (mu.T[:, :NZ], lv.T[:, :NZ])


# speedup vs baseline: 1.0633x; 1.0633x over previous
import jax
import jax.numpy as jnp
from jax.experimental import pallas as pl
from jax.experimental.pallas import tpu as pltpu

B = 256
MAX_N = 64
NVT = 20
HS = 501
NZ = 56
HP = 512
GP = 3 * HP
NP = 128
ZP = 128
PHASES = 8
PLEN = MAX_N // PHASES


def _body(oh_ref, adjP_ref, wih_ref, whh_ref, bhn_ref, wgm_ref, wgmb_ref,
          w1_ref, b1_ref, w2_ref, b2_ref,
          mu_ref, lv_ref, m_ref, hv_ref):
    m_ref[...] = jnp.zeros_like(m_ref)
    vlane = jax.lax.broadcasted_iota(jnp.int32, (1, MAX_N), 1)

    def make_step(pref):
        def step(v, agg):
            vm1 = jnp.maximum(v - 1, 0)
            c1 = adjP_ref[pl.ds(v, 1), pl.ds(vm1, 1)].reshape(1, B)
            mprev = m_ref[pl.ds(vm1, 1)][0]
            h_in = agg + (mprev * c1).astype(jnp.float32)
            onehot = oh_ref[pl.ds(v, 1)][0]
            gi = jnp.dot(wih_ref[...], onehot,
                         preferred_element_type=jnp.float32)
            gh = jnp.dot(whh_ref[...], h_in.astype(jnp.bfloat16),
                         preferred_element_type=jnp.float32)
            r = jax.nn.sigmoid(gi[0:HP] + gh[0:HP])
            z = jax.nn.sigmoid(gi[HP:2 * HP] + gh[HP:2 * HP])
            n = jnp.tanh(gi[2 * HP:GP] + r * (gh[2 * HP:GP] + bhn_ref[...]))
            hv = (1.0 - z) * n + z * h_in
            hv_ref[...] = hv
            vmask = (vlane == v).astype(jnp.float32)
            gmb = jnp.sum(wgmb_ref[...] * vmask, axis=1, keepdims=True)
            gm = jnp.dot(wgm_ref[...], hv.astype(jnp.bfloat16),
                         preferred_element_type=jnp.float32) + gmb
            mv = jax.nn.sigmoid(gm[0:HP]) * gm[HP:2 * HP]
            vp1 = jnp.minimum(v + 1, MAX_N - 1)
            cn = adjP_ref[pl.ds(vp1, 1)][0, 0:pref]
            agg_next = None
            for k0 in range(0, pref, PLEN):
                s = [m_ref[k0 + j] * cn[k0 + j] for j in range(PLEN)]
                t8 = (((s[0] + s[1]) + (s[2] + s[3]))
                      + ((s[4] + s[5]) + (s[6] + s[7])))
                t8 = t8.astype(jnp.float32)
                agg_next = t8 if agg_next is None else agg_next + t8
            m_ref[pl.ds(v, 1)] = mv.astype(jnp.bfloat16)[None]
            return agg_next
        return step

    agg = jnp.zeros((HP, B), jnp.float32)
    for p in range(PHASES):
        agg = jax.lax.fori_loop(p * PLEN, (p + 1) * PLEN,
                                make_step((p + 1) * PLEN), agg)
    hv = hv_ref[...]
    mu_ref[...] = jnp.dot(w1_ref[...], hv,
                          preferred_element_type=jnp.float32) + b1_ref[...]
    lv_ref[...] = jnp.dot(w2_ref[...], hv,
                          preferred_element_type=jnp.float32) + b2_ref[...]


def _pack3(w, cols, dtype):
    out = jnp.zeros((GP, cols), jnp.float32)
    for k in range(3):
        out = out.at[k * HP:k * HP + HS, :w.shape[1]].set(
            w[k * HS:(k + 1) * HS, :])
    return out.astype(dtype)


def kernel(node_types, adj, gru_Wih, gru_Whh, gru_bih, gru_bhh,
           Wg, bg, Wm, W1, b1, W2, b2):
    f32, bf16 = jnp.float32, jnp.bfloat16
    oh = jax.nn.one_hot(node_types.T, NP, axis=1, dtype=bf16)
    adjP = jnp.transpose(adj, (2, 1, 0))[:, :, None, :].astype(bf16)

    bfold = gru_bih + jnp.concatenate(
        [gru_bhh[:HS], gru_bhh[HS:2 * HS], jnp.zeros((HS,), f32)])
    wih = _pack3(gru_Wih + bfold[:, None], NP, bf16)
    whh = _pack3(gru_Whh, HP, bf16)
    bhn = jnp.zeros((HP, 1), f32).at[:HS, 0].set(gru_bhh[2 * HS:])

    wgm = (jnp.zeros((2 * HP, HP), f32)
           .at[:HS, :HS].set(Wg[:, :HS])
           .at[HP:HP + HS, :HS].set(Wm[:, :HS])).astype(bf16)
    wgmb = (jnp.zeros((2 * HP, MAX_N), f32)
            .at[:HS, :].set(bg[:, None] + Wg[:, HS:])
            .at[HP:HP + HS, :].set(Wm[:, HS:]))

    w1 = jnp.zeros((ZP, HP), f32).at[:NZ, :HS].set(W1)
    b1p = jnp.zeros((ZP, 1), f32).at[:NZ, 0].set(b1)
    w2 = jnp.zeros((ZP, HP), f32).at[:NZ, :HS].set(W2)
    b2p = jnp.zeros((ZP, 1), f32).at[:NZ, 0].set(b2)

    mu, lv = pl.pallas_call(
        _body,
        out_shape=(jax.ShapeDtypeStruct((ZP, B), f32),
                   jax.ShapeDtypeStruct((ZP, B), f32)),
        scratch_shapes=[pltpu.VMEM((MAX_N, HP, B), bf16),
                        pltpu.VMEM((HP, B), f32)],
        compiler_params=pltpu.CompilerParams(
            vmem_limit_bytes=120 * 1024 * 1024),
    )(oh, adjP, wih, whh, bhn, wgm, wgmb, w1, b1p, w2, b2p)
    return (mu.T[:, :NZ], lv.T[:, :NZ])


# speedup vs baseline: 1.4172x; 1.3329x over previous
import jax
import jax.numpy as jnp
from jax.experimental import pallas as pl
from jax.experimental.pallas import tpu as pltpu

B = 256
MAX_N = 64
NVT = 20
HS = 501
NZ = 56
HP = 512
GP = 3 * HP
NP = 128
ZP = 128
PHASES = 8
PLEN = MAX_N // PHASES


def _body(oh_ref, adjP_ref, wih_ref, whh_ref, bhn_ref, wgm_ref, wgmb_ref,
          w1_ref, b1_ref, w2_ref, b2_ref,
          mu_ref, lv_ref, m_ref, hv_ref):
    m_ref[...] = jnp.zeros_like(m_ref)
    vlane = jax.lax.broadcasted_iota(jnp.int32, (1, MAX_N), 1)

    def make_step(pref):
        def step(v, carry):
            del carry
            cv = adjP_ref[pl.ds(v, 1)][0, 0:pref]
            h_in = None
            for k0 in range(0, pref, PLEN):
                s = [m_ref[k0 + j] * cv[k0 + j] for j in range(PLEN)]
                t8 = (((s[0] + s[1]) + (s[2] + s[3]))
                      + ((s[4] + s[5]) + (s[6] + s[7])))
                t8 = t8.astype(jnp.float32)
                h_in = t8 if h_in is None else h_in + t8
            onehot = oh_ref[pl.ds(v, 1)][0]
            gi = jnp.dot(wih_ref[...], onehot,
                         preferred_element_type=jnp.float32)
            gh = jnp.dot(whh_ref[...], h_in.astype(jnp.bfloat16),
                         preferred_element_type=jnp.float32)
            r = jax.nn.sigmoid(gi[0:HP] + gh[0:HP])
            z = jax.nn.sigmoid(gi[HP:2 * HP] + gh[HP:2 * HP])
            n = jnp.tanh(gi[2 * HP:GP] + r * (gh[2 * HP:GP] + bhn_ref[...]))
            hv = (1.0 - z) * n + z * h_in
            hv_ref[...] = hv
            vmask = (vlane == v).astype(jnp.float32)
            gmb = jnp.sum(wgmb_ref[...] * vmask, axis=1, keepdims=True)
            gm = jnp.dot(wgm_ref[...], hv.astype(jnp.bfloat16),
                         preferred_element_type=jnp.float32) + gmb
            mv = jax.nn.sigmoid(gm[0:HP]) * gm[HP:2 * HP]
            m_ref[pl.ds(v, 1)] = mv.astype(jnp.bfloat16)[None]
            return 0
        return step

    for p in range(PHASES):
        jax.lax.fori_loop(p * PLEN, (p + 1) * PLEN,
                          make_step((p + 1) * PLEN), 0)
    hv = hv_ref[...]
    mu_ref[...] = jnp.dot(w1_ref[...], hv,
                          preferred_element_type=jnp.float32) + b1_ref[...]
    lv_ref[...] = jnp.dot(w2_ref[...], hv,
                          preferred_element_type=jnp.float32) + b2_ref[...]


def _pack3(w, cols, dtype):
    out = jnp.zeros((GP, cols), jnp.float32)
    for k in range(3):
        out = out.at[k * HP:k * HP + HS, :w.shape[1]].set(
            w[k * HS:(k + 1) * HS, :])
    return out.astype(dtype)


def kernel(node_types, adj, gru_Wih, gru_Whh, gru_bih, gru_bhh,
           Wg, bg, Wm, W1, b1, W2, b2):
    f32, bf16 = jnp.float32, jnp.bfloat16
    oh = jax.nn.one_hot(node_types.T, NP, axis=1, dtype=bf16)
    adjP = jnp.transpose(adj, (2, 1, 0))[:, :, None, :].astype(bf16)

    bfold = gru_bih + jnp.concatenate(
        [gru_bhh[:HS], gru_bhh[HS:2 * HS], jnp.zeros((HS,), f32)])
    wih = _pack3(gru_Wih + bfold[:, None], NP, bf16)
    whh = _pack3(gru_Whh, HP, bf16)
    bhn = jnp.zeros((HP, 1), f32).at[:HS, 0].set(gru_bhh[2 * HS:])

    wgm = (jnp.zeros((2 * HP, HP), f32)
           .at[:HS, :HS].set(Wg[:, :HS])
           .at[HP:HP + HS, :HS].set(Wm[:, :HS])).astype(bf16)
    wgmb = (jnp.zeros((2 * HP, MAX_N), f32)
            .at[:HS, :].set(bg[:, None] + Wg[:, HS:])
            .at[HP:HP + HS, :].set(Wm[:, HS:]))

    w1 = jnp.zeros((ZP, HP), f32).at[:NZ, :HS].set(W1)
    b1p = jnp.zeros((ZP, 1), f32).at[:NZ, 0].set(b1)
    w2 = jnp.zeros((ZP, HP), f32).at[:NZ, :HS].set(W2)
    b2p = jnp.zeros((ZP, 1), f32).at[:NZ, 0].set(b2)

    mu, lv = pl.pallas_call(
        _body,
        out_shape=(jax.ShapeDtypeStruct((ZP, B), f32),
                   jax.ShapeDtypeStruct((ZP, B), f32)),
        scratch_shapes=[pltpu.VMEM((MAX_N, HP, B), bf16),
                        pltpu.VMEM((HP, B), f32)],
        compiler_params=pltpu.CompilerParams(
            vmem_limit_bytes=120 * 1024 * 1024),
    )(oh, adjP, wih, whh, bhn, wgm, wgmb, w1, b1p, w2, b2p)
    return (mu.T[:, :NZ], lv.T[:, :NZ])
